# baseline (speedup 1.0000x reference)
import functools

import numpy as np
import jax
import jax.numpy as jnp
from jax import lax
from jax.experimental import pallas as pl
from jax.experimental.pallas import tpu as pltpu
from jax.experimental.pallas import tpu_sc as plsc

D_MODEL = 768
MAX_SEQ_LEN = 4096
BATCH = 4
SEQ_LEN = 4096
N_ROWS = BATCH * SEQ_LEN

NUM_CORES = 2
NUM_SUBCORES = 16
LANES = 16
NUM_WORKERS = NUM_CORES * NUM_SUBCORES
POS_PER_WORKER = SEQ_LEN // NUM_WORKERS
CHUNK = 16
NUM_CHUNKS = POS_PER_WORKER // CHUNK
PAIRS = BATCH // 2
NUM_STEPS = NUM_CHUNKS * PAIRS
RBUF = 4
LOOKAHEAD = 2

SCALE = float(np.sqrt(np.float32(D_MODEL)))


def _sinusoidal_pe_np(max_seq_len, d_model):
    position = np.arange(0, max_seq_len, dtype=np.float32)[:, None]
    div_term = np.exp(
        np.arange(0, d_model, 2).astype(np.float32) * (-np.log(10000.0) / d_model)
    )
    pe = np.zeros((max_seq_len, d_model), dtype=np.float32)
    pe[:, 0::2] = np.sin(position * div_term)
    pe[:, 1::2] = np.cos(position * div_term)
    return pe


_PE = _sinusoidal_pe_np(MAX_SEQ_LEN, D_MODEL)


_MESH = plsc.VectorSubcoreMesh(core_axis_name="c", subcore_axis_name="s")


@functools.partial(
    pl.kernel,
    mesh=_MESH,
    out_type=jax.ShapeDtypeStruct((N_ROWS, D_MODEL), jnp.float32),
    scratch_types=[
        pltpu.VMEM((BATCH, POS_PER_WORKER), jnp.int32),
        pltpu.VMEM((RBUF, 2, CHUNK, D_MODEL), jnp.float32),
        pltpu.VMEM((2, CHUNK, D_MODEL), jnp.float32),
        pltpu.SemaphoreType.DMA,
        pltpu.SemaphoreType.DMA,
        pltpu.SemaphoreType.DMA,
    ],
)
def _embed_sc(x_hbm, table_hbm, pe_hbm, out_hbm,
              idx_v, rows_v, pe_v, gsem, psem, osem):
    wid = lax.axis_index("s") * NUM_CORES + lax.axis_index("c")
    pos0 = wid * POS_PER_WORKER

    for b in range(BATCH):
        pltpu.sync_copy(
            x_hbm.at[b, pl.ds(pos0, POS_PER_WORKER)], idx_v.at[b])

    def gather_copy(s, b2):
        g = s // PAIRS
        b = (s % PAIRS) * 2 + b2
        return pltpu.make_async_copy(
            table_hbm.at[idx_v.at[b, pl.ds(g * CHUNK, CHUNK)]],
            rows_v.at[s % RBUF, b2], gsem)

    def pe_copy(g):
        return pltpu.make_async_copy(
            pe_hbm.at[pl.ds(pos0 + g * CHUNK, CHUNK)], pe_v.at[g % 2], psem)

    def out_copy(s, b2):
        g = s // PAIRS
        b = (s % PAIRS) * 2 + b2
        return pltpu.make_async_copy(
            rows_v.at[s % RBUF, b2],
            out_hbm.at[pl.ds(b * SEQ_LEN + pos0 + g * CHUNK, CHUNK)], osem)

    for s in range(LOOKAHEAD):
        for b2 in range(2):
            gather_copy(s, b2).start()
    pe_copy(0).start()
    pe_copy(1).start()

    def step_body(s, carry):
        rb = s % RBUF
        g = s // PAIRS
        with jax.named_scope("gwait"):
            for b2 in range(2):
                gather_copy(s, b2).wait()
            @pl.when(s % PAIRS == 0)
            def _():
                pe_copy(0).wait()

        with jax.named_scope("prefetch"):
            @pl.when(s < NUM_STEPS - LOOKAHEAD)
            def _():
                @pl.when(s >= LOOKAHEAD)
                def _():
                    for b2 in range(2):
                        out_copy(0, 0).wait()
                for b2 in range(2):
                    gather_copy(s + LOOKAHEAD, b2).start()

        with jax.named_scope("fma"):
            @plsc.parallel_loop(0, CHUNK, 1, unroll=2)
            def _(r):
                for j in range(D_MODEL // LANES):
                    sl = pl.ds(j * LANES, LANES)
                    pv = pe_v[g % 2, r, sl]
                    for b2 in range(2):
                        rows_v[rb, b2, r, sl] = (
                            pv + rows_v[rb, b2, r, sl] * SCALE)

        with jax.named_scope("ostart"):
            for b2 in range(2):
                out_copy(s, b2).start()
            @pl.when((s % PAIRS == PAIRS - 1) & (g < NUM_CHUNKS - 2))
            def _():
                pe_copy(g + 2).start()
        return carry

    lax.fori_loop(0, NUM_STEPS, step_body, 0)

    for _ in range(2 * RBUF):
        out_copy(0, 0).wait()


def kernel(x, table):
    xf = x.astype(jnp.int32)
    pe = jnp.asarray(_PE)
    out = _embed_sc(xf, table, pe)
    return out.reshape(BATCH, SEQ_LEN, D_MODEL)
